# baseline (speedup 1.0000x reference)
import functools

import jax
import jax.numpy as jnp
from jax import lax
from jax.experimental import pallas as pl
from jax.experimental.pallas import tpu as pltpu
from jax.experimental.pallas import tpu_sc as plsc

N = 10000
E = 320000
C = 128
H = 8
Dh = C // H
NC = 2
NS = 16
NW = NC * NS
EPW = E // NW
K = 80
NCHUNK = EPW // K
NPAD = 10112
RPT = NPAD // NS


BLK0 = 2000


def _proj_body(xu, xi, wu, bu, wi, bi, asu, adu, ast, adt, perm,
               hu_o, hi_o, alsu_o, aldu_o, alst_o, aldt_o):
  del perm
  hu = jnp.dot(xu[...], wu[...], preferred_element_type=jnp.float32) + bu[...]
  hi = jnp.dot(xi[...], wi[...], preferred_element_type=jnp.float32) + bi[...]
  hu_o[...] = hu
  hi_o[...] = hi
  alsu_o[...] = jnp.dot(hu, asu[...], preferred_element_type=jnp.float32)
  aldt_o[...] = jnp.dot(hu, adt[...], preferred_element_type=jnp.float32)
  alst_o[...] = jnp.dot(hi, ast[...], preferred_element_type=jnp.float32)
  aldu_o[...] = jnp.dot(hi, adu[...], preferred_element_type=jnp.float32)


def _project(x_user, x_item, W_user, b_user, W_item, b_item, A_su, A_du,
             A_st, A_dt, P):
  n_blk = N // BLK0
  row = pl.BlockSpec((BLK0, C), lambda i: (i, 0))
  w = pl.BlockSpec((C, C), lambda i: (0, 0))
  b = pl.BlockSpec((1, C), lambda i: (0, 0))
  a = pl.BlockSpec((C, 16), lambda i: (0, 0))
  al = pl.BlockSpec((BLK0, 16), lambda i: (i, 0))
  f32 = jnp.float32
  return pl.pallas_call(
      _proj_body,
      grid=(n_blk,),
      in_specs=[row, row, w, b, w, b, a, a, a, a, w],
      out_specs=[row, row, al, al, al, al],
      out_shape=[
          jax.ShapeDtypeStruct((N, C), f32),
          jax.ShapeDtypeStruct((N, C), f32),
          jax.ShapeDtypeStruct((N, 16), f32),
          jax.ShapeDtypeStruct((N, 16), f32),
          jax.ShapeDtypeStruct((N, 16), f32),
          jax.ShapeDtypeStruct((N, 16), f32),
      ],
  )(x_user, x_item, W_user, b_user.reshape(1, C), W_item,
    b_item.reshape(1, C), A_su, A_du, A_st, A_dt, P)


def _att_mat(att):
  k = lax.broadcasted_iota(jnp.int32, (C, 16), 0)
  hh = lax.broadcasted_iota(jnp.int32, (C, 16), 1)
  return jnp.where(k // Dh == hh, att.reshape(C)[:, None], 0.0)


def _bcast16(v, h):
  idx = jnp.full((16, 1), h, dtype=jnp.int32)
  dnums = lax.GatherDimensionNumbers(
      offset_dims=(), collapsed_slice_dims=(0,), start_index_map=(0,))
  return lax.gather(v, idx, dnums, slice_sizes=(1,),
                    mode=lax.GatherScatterMode.PROMISE_IN_BOUNDS)


def _sc_body(hu, hi, alsu, aldu, alst, aldt, src_ut, dst_ut, src_tu, dst_tu,
             acc_ut, s_ut, acc_tu, s_tu,
             sidx0, didx0, sdidx0, srows0, salv0, dalv0, sbuf0, ebuf0,
             acc_sh, s_sh, sg0, ss0):
  cid = lax.axis_index("c")
  sid = lax.axis_index("s")
  wid = sid * NC + cid
  r0 = sid * RPT
  bufs = (
      (sidx0, didx0, sdidx0, srows0, salv0, dalv0, sbuf0, ebuf0, sg0, ss0),
  )

  for src_e, dst_e, hsrc, als, ald, acc_out, s_out in (
      (src_ut, dst_ut, hu, alsu, aldu, acc_ut, s_ut),
      (src_tu, dst_tu, hi, alst, aldt, acc_tu, s_tu),
  ):
    def zero_body(i, carry):
      z = jnp.zeros((16,), jnp.float32)
      for j in range(C // 16):
        sbuf0[i, pl.ds(j * 16, 16)] = z
      ebuf0[i, :] = z
      return carry

    lax.fori_loop(0, K, zero_body, 0)
    for j in range(RPT // K):
      pltpu.sync_copy(sbuf0.at[pl.ds(0, K)], acc_sh.at[pl.ds(r0 + j * K, K)])
      pltpu.sync_copy(ebuf0.at[pl.ds(0, K)], s_sh.at[pl.ds(r0 + j * K, K)])
    rem = RPT % K
    if rem:
      pltpu.sync_copy(sbuf0.at[pl.ds(0, rem)],
                      acc_sh.at[pl.ds(r0 + (RPT // K) * K, rem)])
      pltpu.sync_copy(ebuf0.at[pl.ds(0, rem)],
                      s_sh.at[pl.ds(r0 + (RPT // K) * K, rem)])
    plsc.subcore_barrier()

    def chunk_body(ci, carry):
      sidx, didx, _, srows, salv, dalv, sbuf, ebuf, sg, sh = bufs[0]
      base = pl.multiple_of(wid * EPW + ci * K, 8)
      pltpu.sync_copy(src_e.at[pl.ds(base, K)], sidx)
      pltpu.sync_copy(dst_e.at[pl.ds(base, K)], didx)
      d1 = pltpu.async_copy(hsrc.at[sidx], srows, sh)
      d2 = pltpu.async_copy(als.at[sidx], salv, sg)
      d3 = pltpu.async_copy(ald.at[didx], dalv, sg)
      d2.wait()
      d3.wait()

      def alpha_body(e, carry2):
        a = salv[e, :] + dalv[e, :]
        a = jnp.maximum(a, 0.2 * a)
        ebuf[e, :] = jnp.exp(a)
        return carry2

      lax.fori_loop(0, K, alpha_body, 0)
      d1.wait()

      def msg_body(e, carry2):
        ev = ebuf[e, :]
        for h in range(H):
          sbuf[e, pl.ds(h * 16, 16)] = srows[e, pl.ds(h * 16, 16)] * _bcast16(ev, h)
        return carry2

      lax.fori_loop(0, K, msg_body, 0, unroll=2)
      pltpu.sync_copy(sbuf, acc_sh.at[didx], add=True)
      pltpu.sync_copy(ebuf, s_sh.at[didx], add=True)
      return carry

    lax.fori_loop(0, NCHUNK, chunk_body, 0)
    plsc.subcore_barrier()

    pltpu.sync_copy(acc_sh.at[pl.ds(r0, RPT)], acc_out.at[cid, pl.ds(r0, RPT)])
    pltpu.sync_copy(s_sh.at[pl.ds(r0, RPT)], s_out.at[cid, pl.ds(r0, RPT)])
    plsc.subcore_barrier()


def _sc_aggregate(hu, hi, alsu, aldu, alst, aldt, src_ut, dst_ut, src_tu,
                  dst_tu):
  f32 = jnp.float32
  mesh = plsc.VectorSubcoreMesh(core_axis_name="c", subcore_axis_name="s")
  out_type = [
      jax.ShapeDtypeStruct((NC, NPAD, C), f32),
      jax.ShapeDtypeStruct((NC, NPAD, 16), f32),
      jax.ShapeDtypeStruct((NC, NPAD, C), f32),
      jax.ShapeDtypeStruct((NC, NPAD, 16), f32),
  ]
  buf_set = [
      pltpu.VMEM((K,), jnp.int32),
      pltpu.VMEM((K,), jnp.int32),
      pltpu.VMEM((K,), jnp.int32),
      pltpu.VMEM((K, C), f32),
      pltpu.VMEM((K, 16), f32),
      pltpu.VMEM((K, 16), f32),
      pltpu.VMEM((K, C), f32),
      pltpu.VMEM((K, 16), f32),
  ]
  scratch = (buf_set + [
      pltpu.VMEM_SHARED((NPAD, C), f32),
      pltpu.VMEM_SHARED((NPAD, 16), f32),
      pltpu.SemaphoreType.DMA,
      pltpu.SemaphoreType.DMA,
  ])
  run = pl.kernel(_sc_body, out_type=out_type, mesh=mesh,
                  scratch_types=scratch,
                  compiler_params=pltpu.CompilerParams(
                      use_tc_tiling_on_sc=False, needs_layout_passes=False))
  return run(hu, hi, alsu, aldu, alst, aldt, src_ut, dst_ut, src_tu, dst_tu)


BLK2 = 2000


def _epi_body(acc_ref, s_ref, out_ref):
  acc = acc_ref[0] + acc_ref[1]
  s = s_ref[0] + s_ref[1]
  j = lax.broadcasted_iota(jnp.int32, (16, C), 0)
  k = lax.broadcasted_iota(jnp.int32, (16, C), 1)
  r = jnp.where(j == k // Dh, 1.0, 0.0)
  srep = jnp.dot(s, r, preferred_element_type=jnp.float32)
  out_ref[...] = jnp.maximum(acc / (srep + 1e-16), 0.0)


def _finish(acc, s):
  n_blk = N // BLK2
  return pl.pallas_call(
      _epi_body,
      grid=(n_blk,),
      in_specs=[
          pl.BlockSpec((NC, BLK2, C), lambda i: (0, i, 0)),
          pl.BlockSpec((NC, BLK2, 16), lambda i: (0, i, 0)),
      ],
      out_specs=pl.BlockSpec((BLK2, C), lambda i: (i, 0)),
      out_shape=jax.ShapeDtypeStruct((N, C), jnp.float32),
  )(acc, s)


def kernel(x_user, x_item, edge_index_ut, edge_index_tu, W_user, b_user,
           W_item, b_item, att_src_ut, att_dst_ut, att_src_tu, att_dst_tu,
           k_W, k_b, q):
  del k_W, k_b, q
  A_su = _att_mat(att_src_ut)
  A_du = _att_mat(att_dst_ut)
  A_st = _att_mat(att_src_tu)
  A_dt = _att_mat(att_dst_tu)
  kout = jnp.arange(C)
  kin = (2 * (kout // 32) + kout % 2) * Dh + (kout % 32) // 2
  P = (jnp.arange(C)[:, None] == kin[None, :]).astype(jnp.float32)
  hu, hi, alsu, aldu, alst, aldt = _project(
      x_user, x_item, W_user, b_user, W_item, b_item, A_su, A_du, A_st, A_dt,
      P)
  ei_ut = edge_index_ut.astype(jnp.int32)
  ei_tu = edge_index_tu.astype(jnp.int32)
  acc_ut, s_ut, acc_tu, s_tu = _sc_aggregate(
      hu, hi, alsu, aldu, alst, aldt, ei_ut[0], ei_ut[1], ei_tu[0], ei_tu[1])
  out_u = _finish(acc_tu, s_tu)
  out_i = _finish(acc_ut, s_ut)
  return jnp.concatenate([out_u, out_i], axis=0)


# speedup vs baseline: 1.7014x; 1.7014x over previous
import functools

import jax
import jax.numpy as jnp
from jax import lax
from jax.experimental import pallas as pl
from jax.experimental.pallas import tpu as pltpu
from jax.experimental.pallas import tpu_sc as plsc

N = 10000
E = 320000
C = 128
H = 8
Dh = C // H
NC = 2
NS = 16
NW = NC * NS
EPW = E // NW
K = 80
NCHUNK = EPW // K
NPAD = 10112
RPT = NPAD // NS


BLK0 = 2000


def _proj_body(xu, xi, wu, bu, wi, bi, asu, adu, ast, adt, perm,
               hu_o, hi_o, alsu_o, aldu_o, alst_o, aldt_o):
  del perm
  hu = jnp.dot(xu[...], wu[...], preferred_element_type=jnp.float32) + bu[...]
  hi = jnp.dot(xi[...], wi[...], preferred_element_type=jnp.float32) + bi[...]
  hu_o[...] = hu
  hi_o[...] = hi
  alsu_o[...] = jnp.dot(hu, asu[...], preferred_element_type=jnp.float32)
  aldt_o[...] = jnp.dot(hu, adt[...], preferred_element_type=jnp.float32)
  alst_o[...] = jnp.dot(hi, ast[...], preferred_element_type=jnp.float32)
  aldu_o[...] = jnp.dot(hi, adu[...], preferred_element_type=jnp.float32)


def _project(x_user, x_item, W_user, b_user, W_item, b_item, A_su, A_du,
             A_st, A_dt, P):
  n_blk = N // BLK0
  row = pl.BlockSpec((BLK0, C), lambda i: (i, 0))
  w = pl.BlockSpec((C, C), lambda i: (0, 0))
  b = pl.BlockSpec((1, C), lambda i: (0, 0))
  a = pl.BlockSpec((C, 16), lambda i: (0, 0))
  al = pl.BlockSpec((BLK0, 16), lambda i: (i, 0))
  f32 = jnp.float32
  return pl.pallas_call(
      _proj_body,
      grid=(n_blk,),
      in_specs=[row, row, w, b, w, b, a, a, a, a, w],
      out_specs=[row, row, al, al, al, al],
      out_shape=[
          jax.ShapeDtypeStruct((N, C), f32),
          jax.ShapeDtypeStruct((N, C), f32),
          jax.ShapeDtypeStruct((N, 16), f32),
          jax.ShapeDtypeStruct((N, 16), f32),
          jax.ShapeDtypeStruct((N, 16), f32),
          jax.ShapeDtypeStruct((N, 16), f32),
      ],
  )(x_user, x_item, W_user, b_user.reshape(1, C), W_item,
    b_item.reshape(1, C), A_su, A_du, A_st, A_dt, P)


def _att_mat(att):
  k = lax.broadcasted_iota(jnp.int32, (C, 16), 0)
  hh = lax.broadcasted_iota(jnp.int32, (C, 16), 1)
  return jnp.where(k // Dh == hh, att.reshape(C)[:, None], 0.0)


def _bcast16(v, h):
  idx = jnp.full((16, 1), h, dtype=jnp.int32)
  dnums = lax.GatherDimensionNumbers(
      offset_dims=(), collapsed_slice_dims=(0,), start_index_map=(0,))
  return lax.gather(v, idx, dnums, slice_sizes=(1,),
                    mode=lax.GatherScatterMode.PROMISE_IN_BOUNDS)


def _sc_body(hu, hi, alsu, aldu, alst, aldt, src_ut, dst_ut, src_tu, dst_tu,
             acc_ut, s_ut, acc_tu, s_tu,
             sidx0, didx0, sdidx0, srows0, salv0, dalv0, sbuf0, ebuf0,
             acc_sh, s_sh, sg0, ss0):
  cid = lax.axis_index("c")
  sid = lax.axis_index("s")
  wid = sid * NC + cid
  r0 = sid * RPT
  bufs = (
      (sidx0, didx0, sdidx0, srows0, salv0, dalv0, sbuf0, ebuf0, sg0, ss0),
  )

  for src_e, dst_e, hsrc, als, ald, acc_out, s_out in (
      (src_ut, dst_ut, hu, alsu, aldu, acc_ut, s_ut),
      (src_tu, dst_tu, hi, alst, aldt, acc_tu, s_tu),
  ):
    def zero_body(i, carry):
      z = jnp.zeros((16,), jnp.float32)
      for j in range(C // 16):
        sbuf0[i, pl.ds(j * 16, 16)] = z
      ebuf0[i, :] = z
      return carry

    lax.fori_loop(0, K, zero_body, 0)
    for j in range(RPT // K):
      pltpu.sync_copy(sbuf0.at[pl.ds(0, K)], acc_sh.at[pl.ds(r0 + j * K, K)])
      pltpu.sync_copy(ebuf0.at[pl.ds(0, K)], s_sh.at[pl.ds(r0 + j * K, K)])
    rem = RPT % K
    if rem:
      pltpu.sync_copy(sbuf0.at[pl.ds(0, rem)],
                      acc_sh.at[pl.ds(r0 + (RPT // K) * K, rem)])
      pltpu.sync_copy(ebuf0.at[pl.ds(0, rem)],
                      s_sh.at[pl.ds(r0 + (RPT // K) * K, rem)])
    plsc.subcore_barrier()

    def chunk_body(ci, carry):
      sidx, didx, _, srows, salv, dalv, sbuf, ebuf, sg, sh = bufs[0]
      base = pl.multiple_of(wid * EPW + ci * K, 8)
      pltpu.sync_copy(src_e.at[pl.ds(base, K)], sidx)
      pltpu.sync_copy(dst_e.at[pl.ds(base, K)], didx)
      d1 = pltpu.async_copy(hsrc.at[sidx], srows, sh)
      d2 = pltpu.async_copy(als.at[sidx], salv, sg)
      d3 = pltpu.async_copy(ald.at[didx], dalv, sg)
      d2.wait()
      d3.wait()

      @plsc.parallel_loop(0, K)
      def alpha_body(e):
        a = salv[e, :] + dalv[e, :]
        a = jnp.maximum(a, 0.2 * a)
        ebuf[e, :] = jnp.exp(a)

      d1.wait()

      @plsc.parallel_loop(0, K)
      def msg_body(e):
        ev = ebuf[e, :]
        for h in range(H):
          sbuf[e, pl.ds(h * 16, 16)] = srows[e, pl.ds(h * 16, 16)] * _bcast16(ev, h)
      pltpu.sync_copy(sbuf, acc_sh.at[didx], add=True)
      pltpu.sync_copy(ebuf, s_sh.at[didx], add=True)
      return carry

    lax.fori_loop(0, NCHUNK, chunk_body, 0)
    plsc.subcore_barrier()

    pltpu.sync_copy(acc_sh.at[pl.ds(r0, RPT)], acc_out.at[cid, pl.ds(r0, RPT)])
    pltpu.sync_copy(s_sh.at[pl.ds(r0, RPT)], s_out.at[cid, pl.ds(r0, RPT)])
    plsc.subcore_barrier()


def _sc_aggregate(hu, hi, alsu, aldu, alst, aldt, src_ut, dst_ut, src_tu,
                  dst_tu):
  f32 = jnp.float32
  mesh = plsc.VectorSubcoreMesh(core_axis_name="c", subcore_axis_name="s")
  out_type = [
      jax.ShapeDtypeStruct((NC, NPAD, C), f32),
      jax.ShapeDtypeStruct((NC, NPAD, 16), f32),
      jax.ShapeDtypeStruct((NC, NPAD, C), f32),
      jax.ShapeDtypeStruct((NC, NPAD, 16), f32),
  ]
  buf_set = [
      pltpu.VMEM((K,), jnp.int32),
      pltpu.VMEM((K,), jnp.int32),
      pltpu.VMEM((K,), jnp.int32),
      pltpu.VMEM((K, C), f32),
      pltpu.VMEM((K, 16), f32),
      pltpu.VMEM((K, 16), f32),
      pltpu.VMEM((K, C), f32),
      pltpu.VMEM((K, 16), f32),
  ]
  scratch = (buf_set + [
      pltpu.VMEM_SHARED((NPAD, C), f32),
      pltpu.VMEM_SHARED((NPAD, 16), f32),
      pltpu.SemaphoreType.DMA,
      pltpu.SemaphoreType.DMA,
  ])
  run = pl.kernel(_sc_body, out_type=out_type, mesh=mesh,
                  scratch_types=scratch,
                  compiler_params=pltpu.CompilerParams(
                      use_tc_tiling_on_sc=False, needs_layout_passes=False))
  return run(hu, hi, alsu, aldu, alst, aldt, src_ut, dst_ut, src_tu, dst_tu)


BLK2 = 2000


def _epi_body(acc_ref, s_ref, out_ref):
  acc = acc_ref[0] + acc_ref[1]
  s = s_ref[0] + s_ref[1]
  j = lax.broadcasted_iota(jnp.int32, (16, C), 0)
  k = lax.broadcasted_iota(jnp.int32, (16, C), 1)
  r = jnp.where(j == k // Dh, 1.0, 0.0)
  srep = jnp.dot(s, r, preferred_element_type=jnp.float32)
  out_ref[...] = jnp.maximum(acc / (srep + 1e-16), 0.0)


def _finish(acc, s):
  n_blk = N // BLK2
  return pl.pallas_call(
      _epi_body,
      grid=(n_blk,),
      in_specs=[
          pl.BlockSpec((NC, BLK2, C), lambda i: (0, i, 0)),
          pl.BlockSpec((NC, BLK2, 16), lambda i: (0, i, 0)),
      ],
      out_specs=pl.BlockSpec((BLK2, C), lambda i: (i, 0)),
      out_shape=jax.ShapeDtypeStruct((N, C), jnp.float32),
  )(acc, s)


def kernel(x_user, x_item, edge_index_ut, edge_index_tu, W_user, b_user,
           W_item, b_item, att_src_ut, att_dst_ut, att_src_tu, att_dst_tu,
           k_W, k_b, q):
  del k_W, k_b, q
  A_su = _att_mat(att_src_ut)
  A_du = _att_mat(att_dst_ut)
  A_st = _att_mat(att_src_tu)
  A_dt = _att_mat(att_dst_tu)
  kout = jnp.arange(C)
  kin = (2 * (kout // 32) + kout % 2) * Dh + (kout % 32) // 2
  P = (jnp.arange(C)[:, None] == kin[None, :]).astype(jnp.float32)
  hu, hi, alsu, aldu, alst, aldt = _project(
      x_user, x_item, W_user, b_user, W_item, b_item, A_su, A_du, A_st, A_dt,
      P)
  ei_ut = edge_index_ut.astype(jnp.int32)
  ei_tu = edge_index_tu.astype(jnp.int32)
  acc_ut, s_ut, acc_tu, s_tu = _sc_aggregate(
      hu, hi, alsu, aldu, alst, aldt, ei_ut[0], ei_ut[1], ei_tu[0], ei_tu[1])
  out_u = _finish(acc_tu, s_tu)
  out_i = _finish(acc_ut, s_ut)
  return jnp.concatenate([out_u, out_i], axis=0)
